# Initial kernel scaffold; baseline (speedup 1.0000x reference)
import jax
import jax.numpy as jnp
from jax.experimental import pallas as pl


def kernel(node_feat, edge_index, edge_attr, W1, b1, W2, b2):
    raise NotImplementedError("write your pallas kernel here")


# speedup vs baseline: 4.0750x; 4.0750x over previous
import functools

import jax
import jax.numpy as jnp
from jax import lax
from jax.experimental import pallas as pl
from jax.experimental.pallas import tpu as pltpu
from jax.experimental.pallas import tpu_sc as plsc

N_NODES = 10000
N_EDGES = 320000
D = 128
LANES = 16
NC = 2
NS = 16
NW = NC * NS
PER_W = N_EDGES // NW
CHUNK = 80
N_CHUNKS = PER_W // CHUNK
GRP = 25
N_GRPS = N_CHUNKS // GRP
ZCHUNKS = (N_NODES + CHUNK - 1) // CHUNK
ZROUNDS = (ZCHUNKS + NS - 1) // NS


def _sc_message_agg(node_feat, src2d, dst2d, edge_attr):
    mesh = plsc.VectorSubcoreMesh(core_axis_name="c", subcore_axis_name="s")

    @functools.partial(
        pl.kernel,
        out_type=jax.ShapeDtypeStruct((NC, N_NODES, D), jnp.float32),
        mesh=mesh,
        scratch_types=[
            pltpu.VMEM((GRP, CHUNK), jnp.int32),
            pltpu.VMEM((GRP, CHUNK), jnp.int32),
            pltpu.VMEM((CHUNK, D), jnp.float32),
            pltpu.VMEM((CHUNK, D), jnp.float32),
            pltpu.VMEM_SHARED((N_NODES, D), jnp.float32),
            pltpu.SemaphoreType.DMA,
        ],
    )
    def k(node_hbm, src_hbm, dst_hbm, ea_hbm, out_hbm,
          src_v, dst_v, rows_v, ea_v, agg_sh, sem):
        cid = lax.axis_index("c")
        sid = lax.axis_index("s")
        wid = sid * NC + cid
        base = wid * PER_W

        @pl.loop(0, CHUNK)
        def _zfill(r):
            for kk in range(D // LANES):
                rows_v[r, pl.ds(kk * LANES, LANES)] = jnp.zeros(
                    (LANES,), jnp.float32)

        @pl.loop(0, ZROUNDS)
        def _zinit(i):
            c = i * NS + sid

            @pl.when(c < ZCHUNKS)
            def _():
                pltpu.sync_copy(rows_v, agg_sh.at[pl.ds(c * CHUNK, CHUNK), :])

        plsc.subcore_barrier()

        @pl.loop(0, N_GRPS)
        def _grp(g):
            pltpu.sync_copy(src_hbm.at[wid, g], src_v)
            pltpu.sync_copy(dst_hbm.at[wid, g], dst_v)

            @pl.loop(0, GRP)
            def _edges(j):
                off = base + (g * GRP + j) * CHUNK
                pltpu.sync_copy(ea_hbm.at[pl.ds(off, CHUNK), :], ea_v)
                pltpu.async_copy(node_hbm.at[src_v.at[j]], rows_v, sem).wait()

                @pl.loop(0, CHUNK)
                def _msg(r):
                    for kk in range(D // LANES):
                        sl = pl.ds(kk * LANES, LANES)
                        rows_v[r, sl] = jnp.maximum(
                            rows_v[r, sl] + ea_v[r, sl], 0.0)

                pltpu.sync_copy(rows_v, agg_sh.at[dst_v.at[j]], add=True)

        plsc.subcore_barrier()

        @pl.when(sid == 0)
        def _dump():
            pltpu.sync_copy(agg_sh, out_hbm.at[cid])

    return k(node_feat, src2d, dst2d, edge_attr)


def _tc_mlp(x, partials, W1, b1, W2, b2):
    n = x.shape[0]
    blk = 1000
    grid = n // blk

    def body(x_ref, p_ref, w1_ref, b1_ref, w2_ref, b2_ref, o_ref):
        h = x_ref[...] + p_ref[0] + p_ref[1]
        h1 = jnp.dot(h, w1_ref[...], preferred_element_type=jnp.float32)
        h1 = jnp.maximum(h1 + b1_ref[...], 0.0)
        h2 = jnp.dot(h1, w2_ref[...], preferred_element_type=jnp.float32)
        o_ref[...] = jnp.maximum(h2 + b2_ref[...], 0.0)

    return pl.pallas_call(
        body,
        grid=(grid,),
        in_specs=[
            pl.BlockSpec((blk, D), lambda i: (i, 0)),
            pl.BlockSpec((NC, blk, D), lambda i: (0, i, 0)),
            pl.BlockSpec((D, 2 * D), lambda i: (0, 0)),
            pl.BlockSpec((1, 2 * D), lambda i: (0, 0)),
            pl.BlockSpec((2 * D, D), lambda i: (0, 0)),
            pl.BlockSpec((1, D), lambda i: (0, 0)),
        ],
        out_specs=pl.BlockSpec((blk, D), lambda i: (i, 0)),
        out_shape=jax.ShapeDtypeStruct((n, D), jnp.float32),
    )(x, partials, W1, b1.reshape(1, -1), W2, b2.reshape(1, -1))


def kernel(node_feat, edge_index, edge_attr, W1, b1, W2, b2):
    src = edge_index[0].astype(jnp.int32).reshape(NW, N_GRPS, GRP, CHUNK)
    dst = edge_index[1].astype(jnp.int32).reshape(NW, N_GRPS, GRP, CHUNK)
    partials = _sc_message_agg(node_feat, src, dst, edge_attr)
    return _tc_mlp(node_feat, partials, W1, b1, W2, b2)


# speedup vs baseline: 7.8278x; 1.9209x over previous
import functools

import jax
import jax.numpy as jnp
from jax import lax
from jax.experimental import pallas as pl
from jax.experimental.pallas import tpu as pltpu
from jax.experimental.pallas import tpu_sc as plsc

N_NODES = 10000
N_EDGES = 320000
D = 128
LANES = 16
NC = 2
NS = 16
NW = NC * NS
PER_W = N_EDGES // NW
CHUNK = 80
N_CHUNKS = PER_W // CHUNK
NBUF = 2
IB = 32
N_CHUNKS_PAD = 128
ZCHUNKS = (N_NODES + CHUNK - 1) // CHUNK
ZROUNDS = (ZCHUNKS + NS - 1) // NS


def _sc_message_agg(node_feat, src2d, dst2d, edge_attr):
    mesh = plsc.VectorSubcoreMesh(core_axis_name="c", subcore_axis_name="s")

    @functools.partial(
        pl.kernel,
        out_type=jax.ShapeDtypeStruct((NC, N_NODES, D), jnp.float32),
        mesh=mesh,
        scratch_types=[
            pltpu.VMEM((IB, CHUNK), jnp.int32),
            pltpu.VMEM((IB, CHUNK), jnp.int32),
            pltpu.VMEM((NBUF, CHUNK, D), jnp.float32),
            pltpu.VMEM((NBUF, CHUNK, D), jnp.float32),
            pltpu.VMEM_SHARED((N_NODES, D), jnp.float32),
            pltpu.SemaphoreType.DMA,
            pltpu.SemaphoreType.DMA,
            pltpu.SemaphoreType.DMA,
            pltpu.SemaphoreType.DMA,
        ],
    )
    def k(node_hbm, src_hbm, dst_hbm, ea_hbm, out_hbm,
          src_v, dst_v, rows_v, ea_v, agg_sh, se0, se1, sg0, sg1):
        cid = lax.axis_index("c")
        sid = lax.axis_index("s")
        wid = sid * NC + cid
        base = wid * PER_W
        sem_e = (se0, se1)
        sem_g = (sg0, sg1)

        @pl.loop(0, CHUNK)
        def _zfill(r):
            for kk in range(D // LANES):
                rows_v[0, r, pl.ds(kk * LANES, LANES)] = jnp.zeros(
                    (LANES,), jnp.float32)

        @pl.loop(0, ZROUNDS)
        def _zinit(i):
            c = i * NS + sid

            @pl.when(c < ZCHUNKS)
            def _():
                pltpu.sync_copy(rows_v.at[0],
                                agg_sh.at[pl.ds(c * CHUNK, CHUNK), :])

        plsc.subcore_barrier()

        def start(jg, jl, b):
            pltpu.async_copy(
                ea_hbm.at[pl.ds(base + jg * CHUNK, CHUNK), :], ea_v.at[b],
                sem_e[b])
            pltpu.async_copy(node_hbm.at[src_v.at[jl]], rows_v.at[b], sem_g[b])

        def wait(jg, jl, b):
            pltpu.make_async_copy(
                ea_hbm.at[pl.ds(base + jg * CHUNK, CHUNK), :], ea_v.at[b],
                sem_e[b]).wait()
            pltpu.make_async_copy(
                node_hbm.at[src_v.at[jl]], rows_v.at[b], sem_g[b]).wait()

        def process(jl, b):
            @pl.loop(0, CHUNK)
            def _msg(r):
                for kk in range(D // LANES):
                    sl = pl.ds(kk * LANES, LANES)
                    rows_v[b, r, sl] = jnp.maximum(
                        rows_v[b, r, sl] + ea_v[b, r, sl], 0.0)

            pltpu.sync_copy(rows_v.at[b], agg_sh.at[dst_v.at[jl]], add=True)

        for off in range(0, N_CHUNKS, IB):
            nb = min(IB, N_CHUNKS - off)
            pltpu.sync_copy(src_hbm.at[wid, pl.ds(off, IB)], src_v)
            pltpu.sync_copy(dst_hbm.at[wid, pl.ds(off, IB)], dst_v)

            for b in range(min(NBUF, nb)):
                start(off + b, b, b)

            even = nb - (nb % NBUF)

            @pl.loop(0, even, step=NBUF)
            def _grp(g):
                for b in range(NBUF):
                    t = g + b
                    wait(off + t, t, b)
                    process(t, b)
                    nxt = t + NBUF

                    @pl.when(nxt < nb)
                    def _():
                        start(off + nxt, nxt, b)

            if nb % NBUF:
                t = nb - 1
                wait(off + t, t, t % NBUF)
                process(t, t % NBUF)

        plsc.subcore_barrier()

        @pl.loop(0, ZROUNDS)
        def _dump(i):
            c = i * NS + sid

            @pl.when(c < ZCHUNKS)
            def _():
                pltpu.sync_copy(agg_sh.at[pl.ds(c * CHUNK, CHUNK), :],
                                out_hbm.at[cid, pl.ds(c * CHUNK, CHUNK), :])

    return k(node_feat, src2d, dst2d, edge_attr)


def _tc_mlp(x, partials, W1, b1, W2, b2):
    n = x.shape[0]
    blk = 1000
    grid = n // blk

    def body(x_ref, p_ref, w1_ref, b1_ref, w2_ref, b2_ref, o_ref):
        h = x_ref[...] + p_ref[0] + p_ref[1]
        h1 = jnp.dot(h, w1_ref[...], preferred_element_type=jnp.float32)
        h1 = jnp.maximum(h1 + b1_ref[...], 0.0)
        h2 = jnp.dot(h1, w2_ref[...], preferred_element_type=jnp.float32)
        o_ref[...] = jnp.maximum(h2 + b2_ref[...], 0.0)

    return pl.pallas_call(
        body,
        grid=(grid,),
        in_specs=[
            pl.BlockSpec((blk, D), lambda i: (i, 0)),
            pl.BlockSpec((NC, blk, D), lambda i: (0, i, 0)),
            pl.BlockSpec((D, 2 * D), lambda i: (0, 0)),
            pl.BlockSpec((1, 2 * D), lambda i: (0, 0)),
            pl.BlockSpec((2 * D, D), lambda i: (0, 0)),
            pl.BlockSpec((1, D), lambda i: (0, 0)),
        ],
        out_specs=pl.BlockSpec((blk, D), lambda i: (i, 0)),
        out_shape=jax.ShapeDtypeStruct((n, D), jnp.float32),
    )(x, partials, W1, b1.reshape(1, -1), W2, b2.reshape(1, -1))


def kernel(node_feat, edge_index, edge_attr, W1, b1, W2, b2):
    pad = ((0, 0), (0, N_CHUNKS_PAD - N_CHUNKS), (0, 0))
    src = jnp.pad(
        edge_index[0].astype(jnp.int32).reshape(NW, N_CHUNKS, CHUNK), pad)
    dst = jnp.pad(
        edge_index[1].astype(jnp.int32).reshape(NW, N_CHUNKS, CHUNK), pad)
    partials = _sc_message_agg(node_feat, src, dst, edge_attr)
    return _tc_mlp(node_feat, partials, W1, b1, W2, b2)


# speedup vs baseline: 7.8405x; 1.0016x over previous
import functools

import jax
import jax.numpy as jnp
from jax import lax
from jax.experimental import pallas as pl
from jax.experimental.pallas import tpu as pltpu
from jax.experimental.pallas import tpu_sc as plsc

N_NODES = 10000
N_EDGES = 320000
D = 128
LANES = 16
NC = 2
NS = 16
NW = NC * NS
PER_W = N_EDGES // NW
CHUNK = 80
N_CHUNKS = PER_W // CHUNK
NBUF = 2
IB = 32
N_CHUNKS_PAD = 128
ZCHUNKS = (N_NODES + CHUNK - 1) // CHUNK
ZROUNDS = (ZCHUNKS + NS - 1) // NS


def _sc_message_agg(node_feat, src2d, dst2d, edge_attr):
    mesh = plsc.VectorSubcoreMesh(core_axis_name="c", subcore_axis_name="s")

    @functools.partial(
        pl.kernel,
        out_type=jax.ShapeDtypeStruct((NC, N_NODES, D), jnp.float32),
        mesh=mesh,
        scratch_types=[
            pltpu.VMEM((IB, CHUNK), jnp.int32),
            pltpu.VMEM((IB, CHUNK), jnp.int32),
            pltpu.VMEM((NBUF, CHUNK, D), jnp.float32),
            pltpu.VMEM((NBUF, CHUNK, D), jnp.float32),
            pltpu.VMEM_SHARED((N_NODES, D), jnp.float32),
            pltpu.SemaphoreType.DMA,
            pltpu.SemaphoreType.DMA,
            pltpu.SemaphoreType.DMA,
            pltpu.SemaphoreType.DMA,
        ],
    )
    def k(node_hbm, src_hbm, dst_hbm, ea_hbm, out_hbm,
          src_v, dst_v, rows_v, ea_v, agg_sh, se0, se1, sg0, sg1):
        cid = lax.axis_index("c")
        sid = lax.axis_index("s")
        wid = sid * NC + cid
        base = wid * PER_W
        sem_e = (se0, se1)
        sem_g = (sg0, sg1)

        @pl.loop(0, CHUNK)
        def _zfill(r):
            for kk in range(D // LANES):
                rows_v[0, r, pl.ds(kk * LANES, LANES)] = jnp.zeros(
                    (LANES,), jnp.float32)

        @pl.loop(0, ZROUNDS)
        def _zinit(i):
            c = i * NS + sid

            @pl.when(c < ZCHUNKS)
            def _():
                pltpu.sync_copy(rows_v.at[0],
                                agg_sh.at[pl.ds(c * CHUNK, CHUNK), :])

        plsc.subcore_barrier()

        def start(jg, jl, b):
            pltpu.async_copy(
                ea_hbm.at[pl.ds(base + jg * CHUNK, CHUNK), :], ea_v.at[b],
                sem_e[b])
            pltpu.async_copy(node_hbm.at[src_v.at[jl]], rows_v.at[b], sem_g[b])

        def wait(jg, jl, b):
            pltpu.make_async_copy(
                ea_hbm.at[pl.ds(base + jg * CHUNK, CHUNK), :], ea_v.at[b],
                sem_e[b]).wait()
            pltpu.make_async_copy(
                node_hbm.at[src_v.at[jl]], rows_v.at[b], sem_g[b]).wait()

        def process(jl, b):
            @pl.loop(0, CHUNK // 4)
            def _msg(r4):
                r = r4 * 4
                for dr in range(4):
                    for kk in range(D // LANES):
                        sl = pl.ds(kk * LANES, LANES)
                        rows_v[b, r + dr, sl] = jnp.maximum(
                            rows_v[b, r + dr, sl] + ea_v[b, r + dr, sl], 0.0)

            pltpu.sync_copy(rows_v.at[b], agg_sh.at[dst_v.at[jl]], add=True)

        for off in range(0, N_CHUNKS, IB):
            nb = min(IB, N_CHUNKS - off)
            pltpu.sync_copy(src_hbm.at[wid, pl.ds(off, IB)], src_v)
            pltpu.sync_copy(dst_hbm.at[wid, pl.ds(off, IB)], dst_v)

            for b in range(min(NBUF, nb)):
                start(off + b, b, b)

            even = nb - (nb % NBUF)

            @pl.loop(0, even, step=NBUF)
            def _grp(g):
                for b in range(NBUF):
                    t = g + b
                    wait(off + t, t, b)
                    process(t, b)
                    nxt = t + NBUF

                    @pl.when(nxt < nb)
                    def _():
                        start(off + nxt, nxt, b)

            if nb % NBUF:
                t = nb - 1
                wait(off + t, t, t % NBUF)
                process(t, t % NBUF)

        plsc.subcore_barrier()

        @pl.loop(0, ZROUNDS)
        def _dump(i):
            c = i * NS + sid

            @pl.when(c < ZCHUNKS)
            def _():
                pltpu.sync_copy(agg_sh.at[pl.ds(c * CHUNK, CHUNK), :],
                                out_hbm.at[cid, pl.ds(c * CHUNK, CHUNK), :])

    return k(node_feat, src2d, dst2d, edge_attr)


def _tc_mlp(x, partials, W1, b1, W2, b2):
    n = x.shape[0]
    blk = 1000
    grid = n // blk

    def body(x_ref, p_ref, w1_ref, b1_ref, w2_ref, b2_ref, o_ref):
        h = x_ref[...] + p_ref[0] + p_ref[1]
        h1 = jnp.dot(h, w1_ref[...], preferred_element_type=jnp.float32)
        h1 = jnp.maximum(h1 + b1_ref[...], 0.0)
        h2 = jnp.dot(h1, w2_ref[...], preferred_element_type=jnp.float32)
        o_ref[...] = jnp.maximum(h2 + b2_ref[...], 0.0)

    return pl.pallas_call(
        body,
        grid=(grid,),
        in_specs=[
            pl.BlockSpec((blk, D), lambda i: (i, 0)),
            pl.BlockSpec((NC, blk, D), lambda i: (0, i, 0)),
            pl.BlockSpec((D, 2 * D), lambda i: (0, 0)),
            pl.BlockSpec((1, 2 * D), lambda i: (0, 0)),
            pl.BlockSpec((2 * D, D), lambda i: (0, 0)),
            pl.BlockSpec((1, D), lambda i: (0, 0)),
        ],
        out_specs=pl.BlockSpec((blk, D), lambda i: (i, 0)),
        out_shape=jax.ShapeDtypeStruct((n, D), jnp.float32),
    )(x, partials, W1, b1.reshape(1, -1), W2, b2.reshape(1, -1))


def kernel(node_feat, edge_index, edge_attr, W1, b1, W2, b2):
    pad = ((0, 0), (0, N_CHUNKS_PAD - N_CHUNKS), (0, 0))
    src = jnp.pad(
        edge_index[0].astype(jnp.int32).reshape(NW, N_CHUNKS, CHUNK), pad)
    dst = jnp.pad(
        edge_index[1].astype(jnp.int32).reshape(NW, N_CHUNKS, CHUNK), pad)
    partials = _sc_message_agg(node_feat, src, dst, edge_attr)
    return _tc_mlp(node_feat, partials, W1, b1, W2, b2)
